# baseline (speedup 1.0000x reference)
import functools

import jax
import jax.numpy as jnp
from jax import lax
from jax.experimental import pallas as pl
from jax.experimental.pallas import tpu as pltpu
from jax.experimental.pallas import tpu_sc as plsc

VOCAB = 1000000
DIM = 64
BATCH = 16384

_INFO = plsc.get_sparse_core_info()
NUM_CORES = _INFO.num_cores
NUM_SUBCORES = _INFO.num_subcores
LANES = _INFO.num_lanes
NUM_WORKERS = NUM_CORES * NUM_SUBCORES
BPW = BATCH // NUM_WORKERS


def _make_sc_dot():
    mesh = plsc.VectorSubcoreMesh(core_axis_name="c", subcore_axis_name="s")

    @functools.partial(
        pl.kernel,
        mesh=mesh,
        compiler_params=pltpu.CompilerParams(use_tc_tiling_on_sc=False),
        out_type=jax.ShapeDtypeStruct((BATCH, LANES), jnp.float32),
        scratch_types=[
            pltpu.VMEM((BPW,), jnp.int32),
            pltpu.VMEM((BPW,), jnp.int32),
            pltpu.VMEM((BPW, DIM), jnp.float32),
            pltpu.VMEM((BPW, DIM), jnp.float32),
            pltpu.VMEM((BPW, LANES), jnp.float32),
            pltpu.SemaphoreType.DMA,
            pltpu.SemaphoreType.DMA,
        ],
    )
    def sc_dot(cw_hbm, xw_hbm, win_hbm, wout_hbm, out_hbm,
               idx_c, idx_x, rows_in, rows_out, part_v, sem_in, sem_out):
        wid = lax.axis_index("s") * NUM_CORES + lax.axis_index("c")
        base = wid * BPW
        pltpu.sync_copy(cw_hbm.at[pl.ds(base, BPW)], idx_c)
        pltpu.sync_copy(xw_hbm.at[pl.ds(base, BPW)], idx_x)
        cp_in = pltpu.async_copy(win_hbm.at[idx_c], rows_in, sem_in)
        cp_out = pltpu.async_copy(wout_hbm.at[idx_x], rows_out, sem_out)
        cp_in.wait()
        cp_out.wait()

        def row_body(r, _):
            acc = jnp.zeros((LANES,), jnp.float32)
            for c in range(DIM // LANES):
                a = rows_in[r, pl.ds(c * LANES, LANES)]
                b = rows_out[r, pl.ds(c * LANES, LANES)]
                acc = acc + a * b
            part_v[r] = acc
            return 0

        lax.fori_loop(0, BPW, row_body, 0)
        pltpu.sync_copy(part_v, out_hbm.at[pl.ds(base, BPW)])

    return sc_dot


_sc_dot = _make_sc_dot()


def _bce_body(part_ref, labels_ref, out_ref):
    part = part_ref[...]
    y = labels_ref[...]
    gi = lax.broadcasted_iota(jnp.int32, (128, 8), 0) // LANES
    gj = lax.broadcasted_iota(jnp.int32, (128, 8), 1)
    fold = (gi == gj).astype(jnp.float32)
    x = jnp.dot(part, fold, preferred_element_type=jnp.float32)
    per = jnp.maximum(x, 0.0) - x * y + jnp.log1p(jnp.exp(-jnp.abs(x)))
    out_ref[0, 0] = jnp.sum(per) * (1.0 / BATCH)


_bce = pl.pallas_call(
    _bce_body,
    out_shape=jax.ShapeDtypeStruct((1, 1), jnp.float32),
    out_specs=pl.BlockSpec(memory_space=pltpu.SMEM),
)


def kernel(center_words, context_words, labels, W_in, W_out):
    part = _sc_dot(center_words.astype(jnp.int32),
                   context_words.astype(jnp.int32), W_in, W_out)
    loss = _bce(part.reshape(BATCH // 8, 128), labels.reshape(BATCH // 8, 8))
    return loss[0, 0]
